# baseline (speedup 1.0000x reference)
import dataclasses

import jax
import jax.numpy as jnp
from jax import lax
from jax.experimental import pallas as pl
from jax.experimental.pallas import tpu as pltpu
from jax.experimental.pallas import tpu_sc as plsc

N = 10000
E = 320000
F_IN = 128
HEADS = 8
HID = 16
NUM_CLASSES = 64

N1 = 10240
CHUNK = 112
NBUF = 2
NTILES = 32
E_TOT = E
_REAL_CH = -(-E_TOT // (NTILES * CHUNK))
NALL = -(-_REAL_CH // NBUF) * NBUF
PER_TILE = NALL * CHUNK
E_PAD = NTILES * PER_TILE

_f32 = jnp.float32
_i32 = jnp.int32

_MESH = plsc.VectorSubcoreMesh(core_axis_name="c", subcore_axis_name="s")

_GDN = lax.GatherDimensionNumbers(
    offset_dims=(), collapsed_slice_dims=(0,), start_index_map=(0,))

_SC_CP = pltpu.CompilerParams()
for _fld, _val in (("needs_layout_passes", False), ("use_tc_tiling_on_sc", False)):
    if _fld in pltpu.CompilerParams.__dataclass_fields__:
        _SC_CP = dataclasses.replace(_SC_CP, **{_fld: _val})


def _make_sc_body(width, fdim):
    heads8 = fdim == 128

    def body(hx_hbm, ad_hbm, sidi_hbm, z_hbm, out_hbm,
             sd0, sdi0, hx0, bv0, sd1, sdi1, hx1, bv1, acc,
             is0, gs0, ss0, is1, gs1, ss1):
        c = lax.axis_index("c")
        s = lax.axis_index("s")
        w = c * 16 + s
        rows = N1 // 16
        pltpu.sync_copy(z_hbm.at[pl.ds(s * rows, rows)],
                        acc.at[pl.ds(s * rows, rows)])
        plsc.subcore_barrier()
        base = w * NALL

        bufs = ((sd0, sdi0, hx0, bv0, is0, gs0, ss0),
                (sd1, sdi1, hx1, bv1, is1, gs1, ss1))

        def idx_start(g, B):
            pltpu.make_async_copy(sidi_hbm.at[base + g], B[0], B[4]).start()

        def idx_wait(B):
            pltpu.make_async_copy(sidi_hbm.at[base], B[0], B[4]).wait()

        def gat_start(B):
            pltpu.make_async_copy(hx_hbm.at[B[0].at[0]], B[2], B[5]).start()
            pltpu.make_async_copy(ad_hbm.at[B[0].at[1]], B[3], B[5]).start()

        def gat_wait(B):
            pltpu.make_async_copy(hx_hbm.at[B[0].at[0]], B[2], B[5]).wait()
            pltpu.make_async_copy(ad_hbm.at[B[0].at[1]], B[3], B[5]).wait()

        def sct_start(B):
            pltpu.make_async_copy(B[2], acc.at[B[1]], B[6]).start(add=True)

        def sct_wait(B):
            pltpu.make_async_copy(B[2], acc.at[B[1]], B[6]).wait()

        def compute(B):
            sdb, sdib, hxb, bvb = B[0], B[1], B[2], B[3]

            @pl.loop(0, CHUNK // 16)
            def _(t):
                sdib[pl.ds(t * 16, 16)] = sdb[1, pl.ds(t * 16, 16)]

            @plsc.parallel_loop(0, CHUNK, unroll=2)
            def _(j):
                al = hxb[j, pl.ds(fdim, 16)] + bvb[j]
                al = jnp.maximum(al, al * 0.2)
                ex = jnp.exp(al)
                hxb[j, pl.ds(fdim, 16)] = ex
                if heads8:
                    for hh in range(8):
                        lane = jnp.full((16, 1), hh, _i32)
                        wv = lax.gather(
                            ex, lane, _GDN, slice_sizes=(1,),
                            mode=lax.GatherScatterMode.PROMISE_IN_BOUNDS)
                        hxb[j, pl.ds(16 * hh, 16)] = \
                            wv * hxb[j, pl.ds(16 * hh, 16)]
                else:
                    for q in range(4):
                        hxb[j, pl.ds(16 * q, 16)] = \
                            ex * hxb[j, pl.ds(16 * q, 16)]

        idx_start(0, bufs[0])
        idx_start(1, bufs[1])
        idx_wait(bufs[0])
        gat_start(bufs[0])

        @pl.loop(0, NALL, step=NBUF)
        def _(g):
            for half in range(NBUF):
                gg = g + half
                B = bufs[half]
                NB = bufs[(half + 1) % NBUF]

                @pl.when(jnp.logical_and(gg >= NBUF - 1, gg + 1 < NALL))
                def _():
                    sct_wait(NB)

                @pl.when(gg + 1 < NALL)
                def _():
                    idx_wait(NB)
                    gat_start(NB)

                gat_wait(B)
                compute(B)
                sct_start(B)

                @pl.when(gg + 2 < NALL)
                def _():
                    idx_start(gg + 2, bufs[(half + 2) % NBUF])

        for bb in range(NBUF):
            sct_wait(bufs[bb])
        plsc.subcore_barrier()
        pltpu.sync_copy(acc.at[pl.ds(s * rows, rows)],
                        out_hbm.at[c, pl.ds(s * rows, rows)])

    return body


def _sc_layer(hx, ad, sidi, zeros, width, fdim):
    k = pl.kernel(
        _make_sc_body(width, fdim),
        out_type=jax.ShapeDtypeStruct((2, N1, width), _f32),
        mesh=_MESH,
        compiler_params=_SC_CP,
        scratch_types=(
            [pltpu.VMEM((2, CHUNK), _i32),
             pltpu.VMEM((CHUNK,), _i32),
             pltpu.VMEM((CHUNK, width), _f32),
             pltpu.VMEM((CHUNK, 16), _f32)] * NBUF
            + [pltpu.VMEM_SHARED((N1, width), _f32)]
            + [pltpu.SemaphoreType.DMA] * (3 * NBUF)
        ),
    )
    return k(hx, ad, sidi, zeros)


_BM = 1024


def _stA_body(x_ref, w_ref, as_ref, ad_ref, e8_ref, hx_ref, b_ref, init_ref):
    h = jnp.dot(x_ref[...], w_ref[...], preferred_element_type=_f32)
    sa = jnp.dot(h, as_ref[...], preferred_element_type=_f32)
    sb = jnp.dot(h, ad_ref[...], preferred_element_type=_f32)
    neg = jnp.full((_BM, 8), -1e30, _f32)
    hx_ref[...] = jnp.concatenate([h, sa, neg], axis=1)
    b_ref[...] = jnp.concatenate([sb, neg], axis=1)
    als = sa + sb
    exs = 0.5 * jnp.exp(jnp.maximum(als, 0.2 * als))
    ex128 = jnp.dot(exs, e8_ref[...], preferred_element_type=_f32)
    init_ref[...] = jnp.concatenate(
        [ex128 * h, exs, jnp.zeros((_BM, 8), _f32)], axis=1)


def _stage_a(x_p, W1, As1, Ad1, E8):
    return pl.pallas_call(
        _stA_body,
        grid=(N1 // _BM,),
        in_specs=[
            pl.BlockSpec((_BM, F_IN), lambda i: (i, 0)),
            pl.BlockSpec((F_IN, 128), lambda i: (0, 0)),
            pl.BlockSpec((128, 8), lambda i: (0, 0)),
            pl.BlockSpec((128, 8), lambda i: (0, 0)),
            pl.BlockSpec((8, 128), lambda i: (0, 0)),
        ],
        out_specs=[
            pl.BlockSpec((_BM, 144), lambda i: (i, 0)),
            pl.BlockSpec((_BM, 16), lambda i: (i, 0)),
            pl.BlockSpec((_BM, 144), lambda i: (i, 0)),
        ],
        out_shape=[
            jax.ShapeDtypeStruct((N1, 144), _f32),
            jax.ShapeDtypeStruct((N1, 16), _f32),
            jax.ShapeDtypeStruct((N1, 144), _f32),
        ],
    )(x_p, W1, As1, Ad1, E8)


def _stB_body(acc_ref, b1_ref, w2_ref, a2s_ref, a2d_ref, e8_ref,
              hx_ref, b_ref, init_ref):
    accs = acc_ref[0] + acc_ref[1]
    msg = accs[:, :128]
    den = accs[:, 128:136]
    den128 = jnp.dot(den, e8_ref[...], preferred_element_type=_f32)
    y = msg / (den128 + 1e-16) + b1_ref[...]
    y = jnp.maximum(y, 0.01 * y)
    h2 = jnp.dot(y, w2_ref[...], preferred_element_type=_f32)
    sa = jnp.dot(h2, a2s_ref[...], preferred_element_type=_f32)
    sb = jnp.dot(h2, a2d_ref[...], preferred_element_type=_f32)
    hx_ref[...] = jnp.concatenate([h2, sa], axis=1)
    b_ref[...] = sb
    als = sa + sb
    exs = 0.5 * jnp.exp(jnp.maximum(als, 0.2 * als))
    init_ref[...] = jnp.concatenate([exs[:, 0:1] * h2, exs], axis=1)


def _stage_b(acc1, b1, W2, A2s, A2d, E8):
    return pl.pallas_call(
        _stB_body,
        grid=(N1 // _BM,),
        in_specs=[
            pl.BlockSpec((2, _BM, 144), lambda i: (0, i, 0)),
            pl.BlockSpec((1, 128), lambda i: (0, 0)),
            pl.BlockSpec((128, 64), lambda i: (0, 0)),
            pl.BlockSpec((64, 16), lambda i: (0, 0)),
            pl.BlockSpec((64, 16), lambda i: (0, 0)),
            pl.BlockSpec((8, 128), lambda i: (0, 0)),
        ],
        out_specs=[
            pl.BlockSpec((_BM, 80), lambda i: (i, 0)),
            pl.BlockSpec((_BM, 16), lambda i: (i, 0)),
            pl.BlockSpec((_BM, 80), lambda i: (i, 0)),
        ],
        out_shape=[
            jax.ShapeDtypeStruct((N1, 80), _f32),
            jax.ShapeDtypeStruct((N1, 16), _f32),
            jax.ShapeDtypeStruct((N1, 80), _f32),
        ],
    )(acc1, b1, W2, A2s, A2d, E8)


def _stC_body(acc_ref, b2_ref, o_ref):
    accs = acc_ref[0] + acc_ref[1]
    msg = accs[:, :64]
    den = accs[:, 64:65]
    logits = msg / (den + 1e-16) + b2_ref[...]
    m = jnp.max(logits, axis=1, keepdims=True)
    e = jnp.exp(logits - m)
    o_ref[...] = e / jnp.sum(e, axis=1, keepdims=True)


def _stage_c(acc2, b2):
    return pl.pallas_call(
        _stC_body,
        grid=(N1 // _BM,),
        in_specs=[
            pl.BlockSpec((2, _BM, 80), lambda i: (0, i, 0)),
            pl.BlockSpec((1, 64), lambda i: (0, 0)),
        ],
        out_specs=pl.BlockSpec((_BM, 64), lambda i: (i, 0)),
        out_shape=jax.ShapeDtypeStruct((N1, 64), _f32),
    )(acc2, b2)


def kernel(x, edge_index, W1, att_src1, att_dst1, b1, W2, att_src2, att_dst2, b2):
    src = edge_index[0].astype(_i32)
    dst = edge_index[1].astype(_i32)
    fill = N + (jnp.arange(E_PAD - E_TOT, dtype=_i32) % (N1 - N))
    si = jnp.concatenate([src, fill])
    di = jnp.concatenate([dst, fill])
    sidi = jnp.stack([si.reshape(NTILES * NALL, CHUNK),
                      di.reshape(NTILES * NALL, CHUNK)], axis=1)

    x_p = jnp.pad(x, ((0, N1 - N), (0, 0)))

    eye8 = jnp.eye(8, dtype=_f32)
    As1 = (att_src1.reshape(8, 16)[:, :, None] * eye8[:, None, :]).reshape(128, 8)
    Ad1 = (att_dst1.reshape(8, 16)[:, :, None] * eye8[:, None, :]).reshape(128, 8)
    E8 = jnp.repeat(eye8, 16, axis=1)
    A2s = jnp.tile(att_src2.reshape(64, 1), (1, 16))
    A2d = jnp.tile(att_dst2.reshape(64, 1), (1, 16))

    hx1, ad1, init1 = _stage_a(x_p, W1, As1, Ad1, E8)
    acc1 = _sc_layer(hx1, ad1, sidi, init1, 144, 128)
    hx2, ad2, init2 = _stage_b(acc1, b1.reshape(1, 128), W2, A2s, A2d, E8)
    acc2 = _sc_layer(hx2, ad2, sidi, init2, 80, 64)
    out = _stage_c(acc2, b2.reshape(1, 64))
    return out[:N]


# speedup vs baseline: 1.0315x; 1.0315x over previous
import dataclasses

import jax
import jax.numpy as jnp
from jax import lax
from jax.experimental import pallas as pl
from jax.experimental.pallas import tpu as pltpu
from jax.experimental.pallas import tpu_sc as plsc

N = 10000
E = 320000
F_IN = 128
HEADS = 8
HID = 16
NUM_CLASSES = 64

N1 = 10240
CHUNK = 112
NBUF = 2
NTILES = 32
E_TOT = E
_REAL_CH = -(-E_TOT // (NTILES * CHUNK))
NALL = -(-_REAL_CH // NBUF) * NBUF
PER_TILE = NALL * CHUNK
E_PAD = NTILES * PER_TILE

_f32 = jnp.float32
_i32 = jnp.int32

_MESH = plsc.VectorSubcoreMesh(core_axis_name="c", subcore_axis_name="s")

_GDN = lax.GatherDimensionNumbers(
    offset_dims=(), collapsed_slice_dims=(0,), start_index_map=(0,))

_SC_CP = pltpu.CompilerParams()
for _fld, _val in (("needs_layout_passes", False), ("use_tc_tiling_on_sc", False)):
    if _fld in pltpu.CompilerParams.__dataclass_fields__:
        _SC_CP = dataclasses.replace(_SC_CP, **{_fld: _val})


def _make_sc_body(width, fdim):
    heads8 = fdim == 128

    def body(hx_hbm, ad_hbm, si_hbm, di_hbm, z_hbm, out_hbm,
             si0, di0, sdi0, hx0, bv0, si1, di1, sdi1, hx1, bv1, acc,
             is0, gs0, ss0, is1, gs1, ss1):
        c = lax.axis_index("c")
        s = lax.axis_index("s")
        w = c * 16 + s
        rows = N1 // 16
        pltpu.sync_copy(z_hbm.at[pl.ds(s * rows, rows)],
                        acc.at[pl.ds(s * rows, rows)])
        plsc.subcore_barrier()
        base = w * PER_TILE

        bufs = ((si0, di0, sdi0, hx0, bv0, is0, gs0, ss0),
                (si1, di1, sdi1, hx1, bv1, is1, gs1, ss1))

        def idx_start(g, B):
            off = base + g * CHUNK
            pltpu.make_async_copy(si_hbm.at[pl.ds(off, CHUNK)], B[0], B[5]).start()
            pltpu.make_async_copy(di_hbm.at[pl.ds(off, CHUNK)], B[1], B[5]).start()

        def idx_wait(B):
            pltpu.make_async_copy(si_hbm.at[pl.ds(base, CHUNK)], B[0], B[5]).wait()
            pltpu.make_async_copy(di_hbm.at[pl.ds(base, CHUNK)], B[1], B[5]).wait()

        def gat_start(B):
            pltpu.make_async_copy(hx_hbm.at[B[0]], B[3], B[6]).start()
            pltpu.make_async_copy(ad_hbm.at[B[1]], B[4], B[6]).start()

        def gat_wait(B):
            pltpu.make_async_copy(hx_hbm.at[B[0]], B[3], B[6]).wait()
            pltpu.make_async_copy(ad_hbm.at[B[1]], B[4], B[6]).wait()

        def sct_start(B):
            pltpu.make_async_copy(B[3], acc.at[B[2]], B[7]).start(add=True)

        def sct_wait(B):
            pltpu.make_async_copy(B[3], acc.at[B[2]], B[7]).wait()

        def compute(B):
            dib, sdib, hxb, bvb = B[1], B[2], B[3], B[4]

            @pl.loop(0, CHUNK // 16)
            def _(t):
                sdib[pl.ds(t * 16, 16)] = dib[pl.ds(t * 16, 16)]

            @plsc.parallel_loop(0, CHUNK, unroll=2)
            def _(j):
                al = hxb[j, pl.ds(fdim, 16)] + bvb[j]
                al = jnp.maximum(al, al * 0.2)
                ex = jnp.exp(al)
                hxb[j, pl.ds(fdim, 16)] = ex
                if heads8:
                    for hh in range(8):
                        lane = jnp.full((16, 1), hh, _i32)
                        wv = lax.gather(
                            ex, lane, _GDN, slice_sizes=(1,),
                            mode=lax.GatherScatterMode.PROMISE_IN_BOUNDS)
                        hxb[j, pl.ds(16 * hh, 16)] = \
                            wv * hxb[j, pl.ds(16 * hh, 16)]
                else:
                    for q in range(4):
                        hxb[j, pl.ds(16 * q, 16)] = \
                            ex * hxb[j, pl.ds(16 * q, 16)]

        idx_start(0, bufs[0])
        idx_start(1, bufs[1])
        idx_wait(bufs[0])
        gat_start(bufs[0])

        @pl.loop(0, NALL, step=NBUF)
        def _(g):
            for half in range(NBUF):
                gg = g + half
                B = bufs[half]
                NB = bufs[(half + 1) % NBUF]

                @pl.when(jnp.logical_and(gg >= NBUF - 1, gg + 1 < NALL))
                def _():
                    sct_wait(NB)

                @pl.when(gg + 1 < NALL)
                def _():
                    idx_wait(NB)
                    gat_start(NB)

                gat_wait(B)
                compute(B)
                sct_start(B)

                @pl.when(gg + 2 < NALL)
                def _():
                    idx_start(gg + 2, bufs[(half + 2) % NBUF])

        for bb in range(NBUF):
            sct_wait(bufs[bb])
        plsc.subcore_barrier()
        pltpu.sync_copy(acc.at[pl.ds(s * rows, rows)],
                        out_hbm.at[c, pl.ds(s * rows, rows)])

    return body


def _sc_layer(hx, ad, si, di, zeros, width, fdim):
    k = pl.kernel(
        _make_sc_body(width, fdim),
        out_type=jax.ShapeDtypeStruct((2, N1, width), _f32),
        mesh=_MESH,
        compiler_params=_SC_CP,
        scratch_types=(
            [pltpu.VMEM((CHUNK,), _i32),
             pltpu.VMEM((CHUNK,), _i32),
             pltpu.VMEM((CHUNK,), _i32),
             pltpu.VMEM((CHUNK, width), _f32),
             pltpu.VMEM((CHUNK, 16), _f32)] * NBUF
            + [pltpu.VMEM_SHARED((N1, width), _f32)]
            + [pltpu.SemaphoreType.DMA] * (3 * NBUF)
        ),
    )
    return k(hx, ad, si, di, zeros)


_BM = 1024


def _stA_body(x_ref, w_ref, as_ref, ad_ref, e8_ref, hx_ref, b_ref, init_ref):
    h = jnp.dot(x_ref[...], w_ref[...], preferred_element_type=_f32)
    sa = jnp.dot(h, as_ref[...], preferred_element_type=_f32)
    sb = jnp.dot(h, ad_ref[...], preferred_element_type=_f32)
    neg = jnp.full((_BM, 8), -1e30, _f32)
    hx_ref[...] = jnp.concatenate([h, sa, neg], axis=1)
    b_ref[...] = jnp.concatenate([sb, neg], axis=1)
    als = sa + sb
    exs = 0.5 * jnp.exp(jnp.maximum(als, 0.2 * als))
    ex128 = jnp.dot(exs, e8_ref[...], preferred_element_type=_f32)
    init_ref[...] = jnp.concatenate(
        [ex128 * h, exs, jnp.zeros((_BM, 8), _f32)], axis=1)


def _stage_a(x_p, W1, As1, Ad1, E8):
    return pl.pallas_call(
        _stA_body,
        grid=(N1 // _BM,),
        in_specs=[
            pl.BlockSpec((_BM, F_IN), lambda i: (i, 0)),
            pl.BlockSpec((F_IN, 128), lambda i: (0, 0)),
            pl.BlockSpec((128, 8), lambda i: (0, 0)),
            pl.BlockSpec((128, 8), lambda i: (0, 0)),
            pl.BlockSpec((8, 128), lambda i: (0, 0)),
        ],
        out_specs=[
            pl.BlockSpec((_BM, 144), lambda i: (i, 0)),
            pl.BlockSpec((_BM, 16), lambda i: (i, 0)),
            pl.BlockSpec((_BM, 144), lambda i: (i, 0)),
        ],
        out_shape=[
            jax.ShapeDtypeStruct((N1, 144), _f32),
            jax.ShapeDtypeStruct((N1, 16), _f32),
            jax.ShapeDtypeStruct((N1, 144), _f32),
        ],
    )(x_p, W1, As1, Ad1, E8)


def _stB_body(acc_ref, b1_ref, w2_ref, a2s_ref, a2d_ref, e8_ref,
              hx_ref, b_ref, init_ref):
    accs = acc_ref[0] + acc_ref[1]
    msg = accs[:, :128]
    den = accs[:, 128:136]
    den128 = jnp.dot(den, e8_ref[...], preferred_element_type=_f32)
    y = msg / (den128 + 1e-16) + b1_ref[...]
    y = jnp.maximum(y, 0.01 * y)
    h2 = jnp.dot(y, w2_ref[...], preferred_element_type=_f32)
    sa = jnp.dot(h2, a2s_ref[...], preferred_element_type=_f32)
    sb = jnp.dot(h2, a2d_ref[...], preferred_element_type=_f32)
    hx_ref[...] = jnp.concatenate([h2, sa], axis=1)
    b_ref[...] = sb
    als = sa + sb
    exs = 0.5 * jnp.exp(jnp.maximum(als, 0.2 * als))
    init_ref[...] = jnp.concatenate([exs[:, 0:1] * h2, exs], axis=1)


def _stage_b(acc1, b1, W2, A2s, A2d, E8):
    return pl.pallas_call(
        _stB_body,
        grid=(N1 // _BM,),
        in_specs=[
            pl.BlockSpec((2, _BM, 144), lambda i: (0, i, 0)),
            pl.BlockSpec((1, 128), lambda i: (0, 0)),
            pl.BlockSpec((128, 64), lambda i: (0, 0)),
            pl.BlockSpec((64, 16), lambda i: (0, 0)),
            pl.BlockSpec((64, 16), lambda i: (0, 0)),
            pl.BlockSpec((8, 128), lambda i: (0, 0)),
        ],
        out_specs=[
            pl.BlockSpec((_BM, 80), lambda i: (i, 0)),
            pl.BlockSpec((_BM, 16), lambda i: (i, 0)),
            pl.BlockSpec((_BM, 80), lambda i: (i, 0)),
        ],
        out_shape=[
            jax.ShapeDtypeStruct((N1, 80), _f32),
            jax.ShapeDtypeStruct((N1, 16), _f32),
            jax.ShapeDtypeStruct((N1, 80), _f32),
        ],
    )(acc1, b1, W2, A2s, A2d, E8)


def _stC_body(acc_ref, b2_ref, o_ref):
    accs = acc_ref[0] + acc_ref[1]
    msg = accs[:, :64]
    den = accs[:, 64:65]
    logits = msg / (den + 1e-16) + b2_ref[...]
    m = jnp.max(logits, axis=1, keepdims=True)
    e = jnp.exp(logits - m)
    o_ref[...] = e / jnp.sum(e, axis=1, keepdims=True)


def _stage_c(acc2, b2):
    return pl.pallas_call(
        _stC_body,
        grid=(N1 // _BM,),
        in_specs=[
            pl.BlockSpec((2, _BM, 80), lambda i: (0, i, 0)),
            pl.BlockSpec((1, 64), lambda i: (0, 0)),
        ],
        out_specs=pl.BlockSpec((_BM, 64), lambda i: (i, 0)),
        out_shape=jax.ShapeDtypeStruct((N1, 64), _f32),
    )(acc2, b2)


def kernel(x, edge_index, W1, att_src1, att_dst1, b1, W2, att_src2, att_dst2, b2):
    src = edge_index[0].astype(_i32)
    dst = edge_index[1].astype(_i32)
    fill = N + (jnp.arange(E_PAD - E_TOT, dtype=_i32) % (N1 - N))
    si = jnp.concatenate([src, fill])
    di = jnp.concatenate([dst, fill])

    x_p = jnp.pad(x, ((0, N1 - N), (0, 0)))

    eye8 = jnp.eye(8, dtype=_f32)
    As1 = (att_src1.reshape(8, 16)[:, :, None] * eye8[:, None, :]).reshape(128, 8)
    Ad1 = (att_dst1.reshape(8, 16)[:, :, None] * eye8[:, None, :]).reshape(128, 8)
    E8 = jnp.repeat(eye8, 16, axis=1)
    A2s = jnp.tile(att_src2.reshape(64, 1), (1, 16))
    A2d = jnp.tile(att_dst2.reshape(64, 1), (1, 16))

    hx1, ad1, init1 = _stage_a(x_p, W1, As1, Ad1, E8)
    acc1 = _sc_layer(hx1, ad1, si, di, init1, 144, 128)
    hx2, ad2, init2 = _stage_b(acc1, b1.reshape(1, 128), W2, A2s, A2d, E8)
    acc2 = _sc_layer(hx2, ad2, si, di, init2, 80, 64)
    out = _stage_c(acc2, b2.reshape(1, 64))
    return out[:N]
